# baseline (speedup 1.0000x reference)
import jax
import jax.numpy as jnp
from jax.experimental import pallas as pl
from jax.experimental.pallas import tpu as pltpu

_CB = 1024
_D = 64
_DE = 72
_BETA = 0.1
_BLOCK_T = 2304


def _vq_body(x_ref, e_ref, vals_ref, idx_ref, loss_ref, e2_ref, ee_ref):
    nb = pl.num_programs(0)
    pid = pl.program_id(0)
    xb = x_ref[...]
    e = e_ref[...]

    @pl.when(pid == 0)
    def _():
        e2_ref[...] = jnp.sum(e * e, axis=1)[None, :]
        iota_c = jax.lax.broadcasted_iota(jnp.int32, (_CB, 1), 0).astype(jnp.float32)
        ee_ref[...] = jnp.concatenate(
            [e, iota_c, jnp.ones((_CB, 1), jnp.float32),
             jnp.zeros((_CB, _DE - _D - 2), jnp.float32)], axis=1)
        loss_ref[0, 0] = 0.0

    xe = jax.lax.dot_general(xb, e, (((1,), (1,)), ((), ())),
                             preferred_element_type=jnp.float32)
    x2 = jnp.sum(xb * xb, axis=1, keepdims=True)
    dist = (x2 + e2_ref[...]) - 2.0 * xe
    minval = jnp.min(dist, axis=1, keepdims=True)
    onehot = (dist == minval).astype(jnp.float32)
    ext = jax.lax.dot_general(onehot, ee_ref[...], (((1,), (0,)), ((), ())),
                              preferred_element_type=jnp.float32)
    vals_ref[...] = ext[:, :_D]
    idx_ref[0, 0, :] = ext[:, _D].astype(jnp.int32)
    loss_ref[0, 0] += jnp.sum(minval)

    @pl.when(jnp.max(ext[:, _D + 1]) > 1.5)
    def _():
        iota_f = jax.lax.broadcasted_iota(jnp.int32, dist.shape, 1).astype(jnp.float32)
        idx_f = jnp.min(jnp.where(dist == minval, iota_f, float(_CB)),
                        axis=1, keepdims=True)
        idx_ref[0, 0, :] = idx_f[:, 0].astype(jnp.int32)
        oh = (iota_f == idx_f).astype(jnp.float32)
        vals_ref[...] = jax.lax.dot_general(oh, e, (((1,), (0,)), ((), ())),
                                            preferred_element_type=jnp.float32)

    @pl.when(pid == nb - 1)
    def _():
        n = nb * _BLOCK_T * _D
        loss_ref[0, 0] *= (1.0 + _BETA) / n


@jax.jit
def _vq(x, embedding):
    b, t, d = x.shape
    nt = b * t
    x2d = x.reshape(nt, d)
    nb = nt // _BLOCK_T
    vals, idx3, loss = pl.pallas_call(
        _vq_body,
        grid=(nb,),
        in_specs=[
            pl.BlockSpec((_BLOCK_T, _D), lambda i: (i, 0)),
            pl.BlockSpec((_CB, _D), lambda i: (0, 0)),
        ],
        out_specs=[
            pl.BlockSpec((_BLOCK_T, _D), lambda i: (i, 0)),
            pl.BlockSpec((1, 1, _BLOCK_T), lambda i: (i, 0, 0)),
            pl.BlockSpec((1, 1), lambda i: (0, 0),
                         memory_space=pltpu.SMEM),
        ],
        out_shape=[
            jax.ShapeDtypeStruct((nt, _D), jnp.float32),
            jax.ShapeDtypeStruct((nb, 1, _BLOCK_T), jnp.int32),
            jax.ShapeDtypeStruct((1, 1), jnp.float32),
        ],
        scratch_shapes=[pltpu.VMEM((1, _CB), jnp.float32),
                        pltpu.VMEM((_CB, _DE), jnp.float32)],
    )(x2d, embedding)
    return (vals.reshape(b, t, d), idx3.reshape(b, t), loss[0, 0])


def kernel(x, embedding):
    return _vq(x, embedding)
